# Initial kernel scaffold; baseline (speedup 1.0000x reference)
import jax
import jax.numpy as jnp
from jax import lax
from jax.experimental import pallas as pl
from jax.experimental.pallas import tpu as pltpu
from jax.experimental.pallas import tpu_sc as plsc

DIM = 32
B_TOTAL = 4096 * 200

_info = plsc.get_sparse_core_info()
NC = _info.num_cores
NS = _info.num_subcores
NW = NC * NS
B_PER_W = B_TOTAL // NW
CHUNK = 3200
NCHUNK = B_PER_W // CHUNK


def _gather_body(idx_hbm, table_hbm, out_hbm, idx_v, rows_v, sem):
    wid = lax.axis_index("s") * NC + lax.axis_index("c")
    base = wid * B_PER_W

    def chunk(g, carry):
        off = base + g * CHUNK
        pltpu.sync_copy(idx_hbm.at[pl.ds(off, CHUNK)], idx_v)
        pltpu.async_copy(table_hbm.at[idx_v], rows_v, sem).wait()
        pltpu.sync_copy(rows_v, out_hbm.at[pl.ds(off, CHUNK)])
        return carry

    lax.fori_loop(0, NCHUNK, chunk, 0)


_mesh = plsc.VectorSubcoreMesh(core_axis_name="c", subcore_axis_name="s")

_gather = pl.kernel(
    _gather_body,
    out_type=jax.ShapeDtypeStruct((B_TOTAL, DIM), jnp.float32),
    mesh=_mesh,
    scratch_types=[
        pltpu.VMEM((CHUNK,), jnp.int32),
        pltpu.VMEM((CHUNK, DIM), jnp.float32),
        pltpu.SemaphoreType.DMA,
    ],
)


@jax.jit
def kernel(data, table):
    idx = data.reshape(B_TOTAL).astype(jnp.int32)
    out = _gather(idx, table)
    return out.reshape(data.shape + (DIM,))


# speedup vs baseline: 1.4957x; 1.4957x over previous
import jax
import jax.numpy as jnp
from jax import lax
from jax.experimental import pallas as pl
from jax.experimental.pallas import tpu as pltpu
from jax.experimental.pallas import tpu_sc as plsc

DIM = 32
B_TOTAL = 4096 * 200

_info = plsc.get_sparse_core_info()
NC = _info.num_cores
NS = _info.num_subcores
NW = NC * NS
B_PER_W = B_TOTAL // NW
CHUNK = 3200
NCHUNK = B_PER_W // CHUNK


def _gather_body(idx_hbm, table_hbm, out_hbm, idx_v, rows_v, sem):
    wid = lax.axis_index("s") * NC + lax.axis_index("c")
    base = wid * B_PER_W

    def chunk(g, carry):
        off = base + g * CHUNK
        pltpu.sync_copy(idx_hbm.at[pl.ds(off, CHUNK)], idx_v)
        pltpu.async_copy(table_hbm.at[idx_v], rows_v, sem).wait()
        pltpu.sync_copy(rows_v, out_hbm.at[pl.ds(off, CHUNK)])
        return carry

    lax.fori_loop(0, NCHUNK, chunk, 0)


_mesh = plsc.VectorSubcoreMesh(core_axis_name="c", subcore_axis_name="s")

_gather = pl.kernel(
    _gather_body,
    out_type=jax.ShapeDtypeStruct((B_TOTAL, DIM), jnp.float32),
    mesh=_mesh,
    scratch_types=[
        pltpu.VMEM((CHUNK,), jnp.int32),
        pltpu.VMEM((CHUNK, DIM), jnp.float32),
        pltpu.SemaphoreType.DMA,
    ],
    compiler_params=pltpu.CompilerParams(use_tc_tiling_on_sc=False),
)


@jax.jit
def kernel(data, table):
    idx = data.reshape(B_TOTAL).astype(jnp.int32)
    out = _gather(idx, table)
    return out.reshape(data.shape + (DIM,))


# speedup vs baseline: 1.5000x; 1.0029x over previous
import jax
import jax.numpy as jnp
from jax import lax
from jax.experimental import pallas as pl
from jax.experimental.pallas import tpu as pltpu
from jax.experimental.pallas import tpu_sc as plsc

DIM = 32
B_TOTAL = 4096 * 200

_info = plsc.get_sparse_core_info()
NC = _info.num_cores
NS = _info.num_subcores
NW = NC * NS
B_PER_W = B_TOTAL // NW
CHUNK = 1280
NCHUNK = B_PER_W // CHUNK
NBUF = 2


def _gather_body(idx_hbm, table_hbm, out_hbm,
                 idx_v, rows0, rows1, gsem0, gsem1, ssem0, ssem1):
    wid = lax.axis_index("s") * NC + lax.axis_index("c")
    base = wid * B_PER_W
    rows = (rows0, rows1)
    gsem = (gsem0, gsem1)
    ssem = (ssem0, ssem1)

    pltpu.sync_copy(idx_hbm.at[pl.ds(base, B_PER_W)], idx_v)

    def start_gather(g, b):
        pltpu.make_async_copy(
            table_hbm.at[idx_v.at[pl.ds(g * CHUNK, CHUNK)]], rows[b], gsem[b]
        ).start()

    def wait_gather(b):
        pltpu.make_async_copy(
            table_hbm.at[idx_v.at[pl.ds(0, CHUNK)]], rows[b], gsem[b]
        ).wait()

    def start_store(g, b):
        pltpu.make_async_copy(
            rows[b], out_hbm.at[pl.ds(base + g * CHUNK, CHUNK)], ssem[b]
        ).start()

    def wait_store(b):
        pltpu.make_async_copy(
            rows[b], out_hbm.at[pl.ds(base, CHUNK)], ssem[b]
        ).wait()

    for b in range(NBUF):
        start_gather(b, b)

    def step(i, carry):
        for b in range(NBUF):
            g = NBUF * i + b
            wait_gather(b)
            start_store(g, b)
            wait_store(b)
            start_gather(g + NBUF, b)
        return carry

    lax.fori_loop(0, (NCHUNK - NBUF) // NBUF, step, 0)

    for b in range(NBUF):
        g = NCHUNK - NBUF + b
        wait_gather(b)
        start_store(g, b)
        wait_store(b)


_mesh = plsc.VectorSubcoreMesh(core_axis_name="c", subcore_axis_name="s")

_gather = pl.kernel(
    _gather_body,
    out_type=jax.ShapeDtypeStruct((B_TOTAL, DIM), jnp.float32),
    mesh=_mesh,
    scratch_types=[
        pltpu.VMEM((B_PER_W,), jnp.int32),
        pltpu.VMEM((CHUNK, DIM), jnp.float32),
        pltpu.VMEM((CHUNK, DIM), jnp.float32),
        pltpu.SemaphoreType.DMA,
        pltpu.SemaphoreType.DMA,
        pltpu.SemaphoreType.DMA,
        pltpu.SemaphoreType.DMA,
    ],
    compiler_params=pltpu.CompilerParams(use_tc_tiling_on_sc=False),
)


@jax.jit
def kernel(data, table):
    idx = data.reshape(B_TOTAL).astype(jnp.int32)
    out = _gather(idx, table)
    return out.reshape(data.shape + (DIM,))
